# baseline (speedup 1.0000x reference)
import functools

import jax
import jax.numpy as jnp
from jax import lax
from jax.experimental import pallas as pl
from jax.experimental.pallas import tpu as pltpu
from jax.experimental.pallas import tpu_sc as plsc

N = 10000
D = 128
E = 320000

NC = 2
NS = 16
NW = NC * NS
L = 16

EPW = E // NW
BLK = 80
NBLK = EPW // BLK
NP = 10240

_mesh = plsc.VectorSubcoreMesh(core_axis_name="c", subcore_axis_name="s")
_sc_params = pltpu.CompilerParams(needs_layout_passes=False)


def _z16():
    return jnp.zeros((L,), jnp.float32)


def _wids():
    cid = lax.axis_index("c")
    sid = lax.axis_index("s")
    return cid, sid, sid * NC + cid


@functools.partial(
    pl.kernel,
    out_type=(
        jax.ShapeDtypeStruct((E,), jnp.float32),
        jax.ShapeDtypeStruct((NC, NP), jnp.float32),
    ),
    mesh=_mesh,
    compiler_params=_sc_params,
    scratch_types=[
        pltpu.VMEM((N,), jnp.float32),
        pltpu.VMEM((EPW,), jnp.int32),
        pltpu.VMEM((EPW,), jnp.int32),
        pltpu.VMEM((EPW,), jnp.float32),
        pltpu.VMEM((EPW,), jnp.float32),
        pltpu.VMEM((BLK,), jnp.float32),
        pltpu.VMEM((BLK,), jnp.int32),
        pltpu.VMEM((BLK,), jnp.float32),
        pltpu.VMEM((BLK,), jnp.int32),
        pltpu.VMEM((640,), jnp.float32),
        pltpu.VMEM_SHARED((NP,), jnp.float32),
        pltpu.SemaphoreType.DMA,
        pltpu.SemaphoreType.DMA,
    ],
)
def _prep1(row_hbm, col_hbm, ev_hbm, deg_hbm, vu_hbm, sp_hbm,
           deg_v, row_c, col_c, ev_c, vu_c, stage_v, idx_v, stage_w, idx_w,
           zbuf, acc_sh, sca, scb):
    cid, sid, wid = _wids()

    def _zb(i, c):
        zbuf[pl.ds(i * L, L)] = _z16()
        return c
    lax.fori_loop(0, 640 // L, _zb, 0)
    pltpu.sync_copy(zbuf, acc_sh.at[pl.ds(sid * 640, 640)])
    plsc.subcore_barrier()

    base = wid * EPW
    pltpu.sync_copy(deg_hbm, deg_v)
    pltpu.sync_copy(row_hbm.at[pl.ds(base, EPW)], row_c)
    pltpu.sync_copy(col_hbm.at[pl.ds(base, EPW)], col_c)
    pltpu.sync_copy(ev_hbm.at[pl.ds(base, EPW)], ev_c)

    def _blk_fill(b, stage, idx):
        off = b * BLK
        for g in range(BLK // L):
            o = off + g * L
            row16 = row_c[pl.ds(o, L)]
            col16 = col_c[pl.ds(o, L)]
            vu16 = plsc.load_gather(deg_v, [row16]) * ev_c[pl.ds(o, L)]
            vu_c[pl.ds(o, L)] = vu16
            stage[pl.ds(g * L, L)] = vu16 * plsc.load_gather(deg_v, [col16])
            idx[pl.ds(g * L, L)] = row16

    _blk_fill(0, stage_v, idx_v)
    pltpu.async_copy(stage_v, acc_sh.at[idx_v], sca, add=True)

    def _blk(i, c):
        b0 = 2 * i
        _blk_fill(b0 + 1, stage_w, idx_w)
        pltpu.async_copy(stage_w, acc_sh.at[idx_w], scb, add=True)
        pltpu.make_async_copy(stage_v, acc_sh.at[idx_v], sca).wait()
        _blk_fill(b0 + 2, stage_v, idx_v)
        pltpu.async_copy(stage_v, acc_sh.at[idx_v], sca, add=True)
        pltpu.make_async_copy(stage_w, acc_sh.at[idx_w], scb).wait()
        return c
    lax.fori_loop(0, (NBLK - 1) // 2, _blk, 0)
    pltpu.make_async_copy(stage_v, acc_sh.at[idx_v], sca).wait()

    pltpu.sync_copy(vu_c, vu_hbm.at[pl.ds(base, EPW)])
    plsc.subcore_barrier()
    pltpu.sync_copy(acc_sh.at[pl.ds(sid * 640, 640)],
                    sp_hbm.at[cid, pl.ds(sid * 640, 640)])


@functools.partial(
    pl.kernel,
    out_type=jax.ShapeDtypeStruct((E,), jnp.float32),
    mesh=_mesh,
    compiler_params=_sc_params,
    scratch_types=[
        pltpu.VMEM((NP,), jnp.float32),
        pltpu.VMEM((NP,), jnp.float32),
        pltpu.VMEM((N,), jnp.float32),
        pltpu.VMEM((N,), jnp.float32),
        pltpu.VMEM((EPW,), jnp.int32),
        pltpu.VMEM((EPW,), jnp.float32),
        pltpu.VMEM((EPW,), jnp.float32),
        pltpu.SemaphoreType.DMA,
    ],
)
def _prep2(sp_hbm, deg_hbm, row_hbm, vu_hbm, vb_hbm,
           s0_v, s1_v, deg_v, dbi_v, row_c, vu_c, vb_c, psem):
    cid, sid, wid = _wids()
    base = wid * EPW
    cp1 = pltpu.async_copy(sp_hbm.at[0], s0_v, psem)
    cp2 = pltpu.async_copy(sp_hbm.at[1], s1_v, psem)
    cp3 = pltpu.async_copy(deg_hbm, deg_v, psem)
    cp4 = pltpu.async_copy(row_hbm.at[pl.ds(base, EPW)], row_c, psem)
    cp5 = pltpu.async_copy(vu_hbm.at[pl.ds(base, EPW)], vu_c, psem)
    cp1.wait(); cp2.wait(); cp3.wait(); cp4.wait(); cp5.wait()

    one = jnp.ones((L,), jnp.float32)

    def _nb(i, c):
        o = i * L
        db = s0_v[pl.ds(o, L)] + s1_v[pl.ds(o, L)] - deg_v[pl.ds(o, L)]
        db = jnp.where(db == 0.0, one, db)
        dbi_v[pl.ds(o, L)] = one / db
        return c
    lax.fori_loop(0, N // L, _nb, 0)

    def _eb(i, c):
        o = i * L
        vb_c[pl.ds(o, L)] = vu_c[pl.ds(o, L)] * plsc.load_gather(
            dbi_v, [row_c[pl.ds(o, L)]])
        return c
    lax.fori_loop(0, EPW // L, _eb, 0)
    pltpu.sync_copy(vb_c, vb_hbm.at[pl.ds(base, EPW)])


ZROWS = 64
CHK = 2000
NCHK = EPW // CHK
CBLK = CHK // BLK


@functools.partial(
    pl.kernel,
    out_type=jax.ShapeDtypeStruct((NC, NP, D), jnp.float32),
    mesh=_mesh,
    compiler_params=_sc_params,
    scratch_types=[
        pltpu.VMEM((CHK,), jnp.float32),
        pltpu.VMEM((CHK,), jnp.int32),
        pltpu.VMEM((CHK,), jnp.int32),
        pltpu.VMEM((BLK,), jnp.int32),
        pltpu.VMEM((BLK,), jnp.int32),
        pltpu.VMEM((BLK,), jnp.int32),
        pltpu.VMEM((BLK,), jnp.int32),
        pltpu.VMEM((BLK,), jnp.int32),
        pltpu.VMEM((BLK,), jnp.int32),
        pltpu.VMEM((BLK, D), jnp.float32),
        pltpu.VMEM((BLK, D), jnp.float32),
        pltpu.VMEM((BLK, D), jnp.float32),
        pltpu.VMEM((ZROWS, D), jnp.float32),
        pltpu.VMEM_SHARED((NP, D), jnp.float32),
        pltpu.SemaphoreType.DMA,
        pltpu.SemaphoreType.DMA,
        pltpu.SemaphoreType.DMA,
        pltpu.SemaphoreType.DMA,
        pltpu.SemaphoreType.DMA,
        pltpu.SemaphoreType.DMA,
    ],
)
def _spmm(vals_hbm, row_hbm, col_hbm, x_hbm, part_hbm,
          vals_c, row_c, col_c, gi0, gi1, gi2, si0, si1, si2,
          rows0, rows1, rows2, zbuf, acc_sh,
          g0, g1, g2, s0, s1, s2):
    cid, sid, wid = _wids()

    def _zr(i, c):
        for d in range(D // L):
            zbuf[i, pl.ds(d * L, L)] = _z16()
        return c
    lax.fori_loop(0, ZROWS, _zr, 0)

    def _zc(r, c):
        pltpu.sync_copy(
            zbuf, acc_sh.at[pl.ds(sid * 640 + r * ZROWS, ZROWS), :])
        return c
    lax.fori_loop(0, 640 // ZROWS, _zc, 0)
    plsc.subcore_barrier()

    bufs = ((gi0, si0, rows0, g0, s0),
            (gi1, si1, rows1, g1, s1),
            (gi2, si2, rows2, g2, s2))

    def _fillg(b, t):
        gi, si, rows, gs, ss = t
        off = b * BLK
        for g in range(BLK // L):
            gi[pl.ds(g * L, L)] = col_c[pl.ds(off + g * L, L)]
            si[pl.ds(g * L, L)] = row_c[pl.ds(off + g * L, L)]
        pltpu.async_copy(x_hbm.at[gi], rows, gs)

    def _waitg(t):
        gi, si, rows, gs, ss = t
        pltpu.make_async_copy(x_hbm.at[gi], rows, gs).wait()

    def _scat(t):
        gi, si, rows, gs, ss = t
        pltpu.async_copy(rows, acc_sh.at[si], ss, add=True)

    def _waits(t):
        gi, si, rows, gs, ss = t
        pltpu.make_async_copy(rows, acc_sh.at[si], ss).wait()

    _dn = lax.GatherDimensionNumbers(
        offset_dims=(), collapsed_slice_dims=(0,), start_index_map=(0,))

    def _scale(b, t):
        gi, si, rows, gs, ss = t
        off0 = b * BLK

        def _sg(g, c):
            v16 = vals_c[pl.ds(off0 + g * L, L)]

            @plsc.parallel_loop(0, L, unroll=8)
            def _se(j):
                e = g * L + j
                sp = lax.gather(
                    v16, jnp.full((L, 1), j, jnp.int32), _dn, (1,),
                    mode=lax.GatherScatterMode.PROMISE_IN_BOUNDS)
                for d in range(D // L):
                    rows[e, pl.ds(d * L, L)] = rows[e, pl.ds(d * L, L)] * sp
            return c
        lax.fori_loop(0, BLK // L, _sg, 0)

    def _chunk(cc, c):
        cbase = wid * EPW + cc * CHK
        pltpu.sync_copy(vals_hbm.at[pl.ds(cbase, CHK)], vals_c)
        pltpu.sync_copy(row_hbm.at[pl.ds(cbase, CHK)], row_c)
        pltpu.sync_copy(col_hbm.at[pl.ds(cbase, CHK)], col_c)

        _fillg(0, bufs[0])
        _fillg(1, bufs[1])
        _fillg(2, bufs[2])
        _waitg(bufs[0])
        _scale(0, bufs[0])
        _scat(bufs[0])

        def _tri(i, c2):
            for j in range(3):
                b = 3 * i + 1 + j
                k = (1 + j) % 3
                k2 = j % 3
                _waits(bufs[k2])
                _fillg(b + 2, bufs[k2])
                _waitg(bufs[k])
                _scale(b, bufs[k])
                _scat(bufs[k])
            return c2
        lax.fori_loop(0, (CBLK - 4) // 3, _tri, 0)

        _waits(bufs[0])
        _fillg(24, bufs[0])
        _waitg(bufs[1])
        _scale(22, bufs[1])
        _scat(bufs[1])
        _waitg(bufs[2])
        _scale(23, bufs[2])
        _scat(bufs[2])
        _waitg(bufs[0])
        _scale(24, bufs[0])
        _scat(bufs[0])
        _waits(bufs[1])
        _waits(bufs[2])
        _waits(bufs[0])
        return c
    lax.fori_loop(0, NCHK, _chunk, 0)

    plsc.subcore_barrier()
    pltpu.sync_copy(acc_sh.at[pl.ds(sid * 640, 640), :],
                    part_hbm.at[cid, pl.ds(sid * 640, 640), :])


RB = 1000


def _wf_body(x_ref, wt_ref, b_ref, o_ref):
    o_ref[...] = jnp.dot(
        x_ref[...], wt_ref[...], preferred_element_type=jnp.float32
    ) + b_ref[...]


def _linear(x, wt, b2):
    return pl.pallas_call(
        _wf_body,
        grid=(N // RB,),
        in_specs=[
            pl.BlockSpec((RB, D), lambda i: (i, 0)),
            pl.BlockSpec((D, D), lambda i: (0, 0)),
            pl.BlockSpec((1, D), lambda i: (0, 0)),
        ],
        out_specs=pl.BlockSpec((RB, D), lambda i: (i, 0)),
        out_shape=jax.ShapeDtypeStruct((N, D), jnp.float32),
    )(x, wt, b2)


def _elem_body(a_ref, b_ref, wf_ref, dg_ref, o_ref):
    dg = dg_ref[...]
    dg = jnp.where(dg == 0.0, 1.0, dg)
    t2 = (a_ref[...] + b_ref[...]) / dg
    wfv = wf_ref[...]
    o_ref[...] = wfv * (t2 - wfv)


def _elem(a, b, wf, dg2):
    return pl.pallas_call(
        _elem_body,
        grid=(N // RB,),
        in_specs=[
            pl.BlockSpec((RB, D), lambda i: (i, 0)),
            pl.BlockSpec((RB, D), lambda i: (i, 0)),
            pl.BlockSpec((RB, D), lambda i: (i, 0)),
            pl.BlockSpec((RB, 1), lambda i: (i, 0)),
        ],
        out_specs=pl.BlockSpec((RB, D), lambda i: (i, 0)),
        out_shape=jax.ShapeDtypeStruct((N, D), jnp.float32),
    )(a, b, wf, dg2)


def _add_body(a_ref, b_ref, o_ref):
    o_ref[...] = a_ref[...] + b_ref[...]


def _combine(a, b):
    return pl.pallas_call(
        _add_body,
        grid=(N // RB,),
        in_specs=[
            pl.BlockSpec((RB, D), lambda i: (i, 0)),
            pl.BlockSpec((RB, D), lambda i: (i, 0)),
        ],
        out_specs=pl.BlockSpec((RB, D), lambda i: (i, 0)),
        out_shape=jax.ShapeDtypeStruct((N, D), jnp.float32),
    )(a, b)


def kernel(node_features, edge_index, edge_values, degree, W, b):
    row = edge_index[0].astype(jnp.int32)
    col = edge_index[1].astype(jnp.int32)
    ev = edge_values.astype(jnp.float32)
    deg = degree.astype(jnp.float32)

    wf = _linear(node_features, W.T, b.reshape(1, D))
    vu, sp = _prep1(row, col, ev, deg)
    vb = _prep2(sp, deg, row, vu)

    t1p = _spmm(vu, row, col, wf)
    temp1 = _combine(t1p[0, :N], t1p[1, :N])
    t2p = _spmm(vb, row, col, temp1)
    sh = _elem(t2p[0, :N], t2p[1, :N], wf, deg.reshape(N, 1))
    op = _spmm(ev, row, col, sh)
    return _combine(op[0, :N], op[1, :N])


# speedup vs baseline: 1.0332x; 1.0332x over previous
import functools

import jax
import jax.numpy as jnp
from jax import lax
from jax.experimental import pallas as pl
from jax.experimental.pallas import tpu as pltpu
from jax.experimental.pallas import tpu_sc as plsc

N = 10000
D = 128
E = 320000

NC = 2
NS = 16
NW = NC * NS
L = 16

EPW = E // NW
BLK = 80
NBLK = EPW // BLK
NP = 10240

_mesh = plsc.VectorSubcoreMesh(core_axis_name="c", subcore_axis_name="s")
_sc_params = pltpu.CompilerParams(needs_layout_passes=False)


def _z16():
    return jnp.zeros((L,), jnp.float32)


def _wids():
    cid = lax.axis_index("c")
    sid = lax.axis_index("s")
    return cid, sid, sid * NC + cid


@functools.partial(
    pl.kernel,
    out_type=(
        jax.ShapeDtypeStruct((E,), jnp.float32),
        jax.ShapeDtypeStruct((NC, NP), jnp.float32),
    ),
    mesh=_mesh,
    compiler_params=_sc_params,
    scratch_types=[
        pltpu.VMEM((N,), jnp.float32),
        pltpu.VMEM((EPW,), jnp.int32),
        pltpu.VMEM((EPW,), jnp.int32),
        pltpu.VMEM((EPW,), jnp.float32),
        pltpu.VMEM((EPW,), jnp.float32),
        pltpu.VMEM((128,), jnp.float32),
        pltpu.VMEM((128,), jnp.int32),
        pltpu.VMEM((128,), jnp.float32),
        pltpu.VMEM((128,), jnp.int32),
        pltpu.VMEM((L,), jnp.float32),
        pltpu.VMEM((L,), jnp.int32),
        pltpu.VMEM((640,), jnp.float32),
        pltpu.VMEM_SHARED((NP,), jnp.float32),
        pltpu.SemaphoreType.DMA,
        pltpu.SemaphoreType.DMA,
    ],
)
def _prep1(row_hbm, col_hbm, ev_hbm, deg_hbm, vu_hbm, sp_hbm,
           deg_v, row_c, col_c, ev_c, vu_c, stage_v, idx_v, stage_w, idx_w,
           stage_t, idx_t, zbuf, acc_sh, sca, scb):
    cid, sid, wid = _wids()

    def _zb(i, c):
        zbuf[pl.ds(i * L, L)] = _z16()
        return c
    lax.fori_loop(0, 640 // L, _zb, 0)
    pltpu.sync_copy(zbuf, acc_sh.at[pl.ds(sid * 640, 640)])
    plsc.subcore_barrier()

    base = wid * EPW
    pltpu.sync_copy(deg_hbm, deg_v)
    pltpu.sync_copy(row_hbm.at[pl.ds(base, EPW)], row_c)
    pltpu.sync_copy(col_hbm.at[pl.ds(base, EPW)], col_c)
    pltpu.sync_copy(ev_hbm.at[pl.ds(base, EPW)], ev_c)

    B1 = 128
    NB1 = (EPW // B1) - 1

    def _blk_fill(b, stage, idx):
        off = b * B1
        for g in range(B1 // L):
            o = off + g * L
            row16 = row_c[pl.ds(o, L)]
            col16 = col_c[pl.ds(o, L)]
            vu16 = plsc.load_gather(deg_v, [row16]) * ev_c[pl.ds(o, L)]
            vu_c[pl.ds(o, L)] = vu16
            stage[pl.ds(g * L, L)] = vu16 * plsc.load_gather(deg_v, [col16])
            idx[pl.ds(g * L, L)] = row16

    _blk_fill(0, stage_v, idx_v)
    pltpu.async_copy(stage_v, acc_sh.at[idx_v], sca, add=True)

    def _blk(i, c):
        b0 = 2 * i
        _blk_fill(b0 + 1, stage_w, idx_w)
        pltpu.async_copy(stage_w, acc_sh.at[idx_w], scb, add=True)
        pltpu.make_async_copy(stage_v, acc_sh.at[idx_v], sca).wait()
        _blk_fill(b0 + 2, stage_v, idx_v)
        pltpu.async_copy(stage_v, acc_sh.at[idx_v], sca, add=True)
        pltpu.make_async_copy(stage_w, acc_sh.at[idx_w], scb).wait()
        return c
    lax.fori_loop(0, NB1 // 2, _blk, 0)
    _blk_fill(77, stage_w, idx_w)
    pltpu.async_copy(stage_w, acc_sh.at[idx_w], scb, add=True)
    o = 78 * B1
    row16 = row_c[pl.ds(o, L)]
    vu16 = plsc.load_gather(deg_v, [row16]) * ev_c[pl.ds(o, L)]
    vu_c[pl.ds(o, L)] = vu16
    stage_t[...] = vu16 * plsc.load_gather(deg_v, [col_c[pl.ds(o, L)]])
    idx_t[...] = row16
    pltpu.make_async_copy(stage_v, acc_sh.at[idx_v], sca).wait()
    pltpu.sync_copy(stage_t, acc_sh.at[idx_t], add=True)
    pltpu.make_async_copy(stage_w, acc_sh.at[idx_w], scb).wait()

    pltpu.sync_copy(vu_c, vu_hbm.at[pl.ds(base, EPW)])
    plsc.subcore_barrier()
    pltpu.sync_copy(acc_sh.at[pl.ds(sid * 640, 640)],
                    sp_hbm.at[cid, pl.ds(sid * 640, 640)])


@functools.partial(
    pl.kernel,
    out_type=jax.ShapeDtypeStruct((E,), jnp.float32),
    mesh=_mesh,
    compiler_params=_sc_params,
    scratch_types=[
        pltpu.VMEM((NP,), jnp.float32),
        pltpu.VMEM((NP,), jnp.float32),
        pltpu.VMEM((N,), jnp.float32),
        pltpu.VMEM((N,), jnp.float32),
        pltpu.VMEM((EPW,), jnp.int32),
        pltpu.VMEM((EPW,), jnp.float32),
        pltpu.VMEM((EPW,), jnp.float32),
        pltpu.SemaphoreType.DMA,
    ],
)
def _prep2(sp_hbm, deg_hbm, row_hbm, vu_hbm, vb_hbm,
           s0_v, s1_v, deg_v, dbi_v, row_c, vu_c, vb_c, psem):
    cid, sid, wid = _wids()
    base = wid * EPW
    cp1 = pltpu.async_copy(sp_hbm.at[0], s0_v, psem)
    cp2 = pltpu.async_copy(sp_hbm.at[1], s1_v, psem)
    cp3 = pltpu.async_copy(deg_hbm, deg_v, psem)
    cp4 = pltpu.async_copy(row_hbm.at[pl.ds(base, EPW)], row_c, psem)
    cp5 = pltpu.async_copy(vu_hbm.at[pl.ds(base, EPW)], vu_c, psem)
    cp1.wait(); cp2.wait(); cp3.wait(); cp4.wait(); cp5.wait()

    one = jnp.ones((L,), jnp.float32)

    def _nb(i, c):
        o = i * L
        db = s0_v[pl.ds(o, L)] + s1_v[pl.ds(o, L)] - deg_v[pl.ds(o, L)]
        db = jnp.where(db == 0.0, one, db)
        dbi_v[pl.ds(o, L)] = one / db
        return c
    lax.fori_loop(0, N // L, _nb, 0)

    def _eb(i, c):
        o = i * L
        vb_c[pl.ds(o, L)] = vu_c[pl.ds(o, L)] * plsc.load_gather(
            dbi_v, [row_c[pl.ds(o, L)]])
        return c
    lax.fori_loop(0, EPW // L, _eb, 0)
    pltpu.sync_copy(vb_c, vb_hbm.at[pl.ds(base, EPW)])


ZROWS = 64
CHK = 2000
NCHK = EPW // CHK
CBLK = CHK // BLK


@functools.partial(
    pl.kernel,
    out_type=jax.ShapeDtypeStruct((NC, NP, D), jnp.float32),
    mesh=_mesh,
    compiler_params=_sc_params,
    scratch_types=[
        pltpu.VMEM((CHK,), jnp.float32),
        pltpu.VMEM((CHK,), jnp.int32),
        pltpu.VMEM((CHK,), jnp.int32),
        pltpu.VMEM((BLK,), jnp.int32),
        pltpu.VMEM((BLK,), jnp.int32),
        pltpu.VMEM((BLK,), jnp.int32),
        pltpu.VMEM((BLK,), jnp.int32),
        pltpu.VMEM((BLK,), jnp.int32),
        pltpu.VMEM((BLK,), jnp.int32),
        pltpu.VMEM((BLK, D), jnp.float32),
        pltpu.VMEM((BLK, D), jnp.float32),
        pltpu.VMEM((BLK, D), jnp.float32),
        pltpu.VMEM((ZROWS, D), jnp.float32),
        pltpu.VMEM_SHARED((NP, D), jnp.float32),
        pltpu.SemaphoreType.DMA,
        pltpu.SemaphoreType.DMA,
        pltpu.SemaphoreType.DMA,
        pltpu.SemaphoreType.DMA,
        pltpu.SemaphoreType.DMA,
        pltpu.SemaphoreType.DMA,
    ],
)
def _spmm(vals_hbm, row_hbm, col_hbm, x_hbm, part_hbm,
          vals_c, row_c, col_c, gi0, gi1, gi2, si0, si1, si2,
          rows0, rows1, rows2, zbuf, acc_sh,
          g0, g1, g2, s0, s1, s2):
    cid, sid, wid = _wids()

    def _zr(i, c):
        for d in range(D // L):
            zbuf[i, pl.ds(d * L, L)] = _z16()
        return c
    lax.fori_loop(0, ZROWS, _zr, 0)

    def _zc(r, c):
        pltpu.sync_copy(
            zbuf, acc_sh.at[pl.ds(sid * 640 + r * ZROWS, ZROWS), :])
        return c
    lax.fori_loop(0, 640 // ZROWS, _zc, 0)
    plsc.subcore_barrier()

    bufs = ((gi0, si0, rows0, g0, s0),
            (gi1, si1, rows1, g1, s1),
            (gi2, si2, rows2, g2, s2))

    def _fillg(b, t):
        gi, si, rows, gs, ss = t
        off = b * BLK
        for g in range(BLK // L):
            gi[pl.ds(g * L, L)] = col_c[pl.ds(off + g * L, L)]
            si[pl.ds(g * L, L)] = row_c[pl.ds(off + g * L, L)]
        pltpu.async_copy(x_hbm.at[gi], rows, gs)

    def _waitg(t):
        gi, si, rows, gs, ss = t
        pltpu.make_async_copy(x_hbm.at[gi], rows, gs).wait()

    def _scat(t):
        gi, si, rows, gs, ss = t
        pltpu.async_copy(rows, acc_sh.at[si], ss, add=True)

    def _waits(t):
        gi, si, rows, gs, ss = t
        pltpu.make_async_copy(rows, acc_sh.at[si], ss).wait()

    _dn = lax.GatherDimensionNumbers(
        offset_dims=(), collapsed_slice_dims=(0,), start_index_map=(0,))

    def _scale(b, t):
        gi, si, rows, gs, ss = t
        off0 = b * BLK

        def _sg(g, c):
            v16 = vals_c[pl.ds(off0 + g * L, L)]

            @plsc.parallel_loop(0, L, unroll=4)
            def _se(j):
                e = g * L + j
                sp = lax.gather(
                    v16, jnp.full((L, 1), j, jnp.int32), _dn, (1,),
                    mode=lax.GatherScatterMode.PROMISE_IN_BOUNDS)
                for d in range(D // L):
                    rows[e, pl.ds(d * L, L)] = rows[e, pl.ds(d * L, L)] * sp
            return c
        lax.fori_loop(0, BLK // L, _sg, 0)

    def _chunk(cc, c):
        cbase = wid * EPW + cc * CHK
        pltpu.sync_copy(vals_hbm.at[pl.ds(cbase, CHK)], vals_c)
        pltpu.sync_copy(row_hbm.at[pl.ds(cbase, CHK)], row_c)
        pltpu.sync_copy(col_hbm.at[pl.ds(cbase, CHK)], col_c)

        _fillg(0, bufs[0])
        _fillg(1, bufs[1])
        _fillg(2, bufs[2])
        _waitg(bufs[0])
        _scale(0, bufs[0])
        _scat(bufs[0])

        def _tri(i, c2):
            for j in range(3):
                b = 3 * i + 1 + j
                k = (1 + j) % 3
                k2 = j % 3
                _waits(bufs[k2])
                _fillg(b + 2, bufs[k2])
                _waitg(bufs[k])
                _scale(b, bufs[k])
                _scat(bufs[k])
            return c2
        lax.fori_loop(0, (CBLK - 4) // 3, _tri, 0)

        _waits(bufs[0])
        _fillg(24, bufs[0])
        _waitg(bufs[1])
        _scale(22, bufs[1])
        _scat(bufs[1])
        _waitg(bufs[2])
        _scale(23, bufs[2])
        _scat(bufs[2])
        _waitg(bufs[0])
        _scale(24, bufs[0])
        _scat(bufs[0])
        _waits(bufs[1])
        _waits(bufs[2])
        _waits(bufs[0])
        return c
    lax.fori_loop(0, NCHK, _chunk, 0)

    plsc.subcore_barrier()
    pltpu.sync_copy(acc_sh.at[pl.ds(sid * 640, 640), :],
                    part_hbm.at[cid, pl.ds(sid * 640, 640), :])


RB = 1000


def _wf_body(x_ref, wt_ref, b_ref, o_ref):
    o_ref[...] = jnp.dot(
        x_ref[...], wt_ref[...], preferred_element_type=jnp.float32
    ) + b_ref[...]


def _linear(x, wt, b2):
    return pl.pallas_call(
        _wf_body,
        grid=(N // RB,),
        in_specs=[
            pl.BlockSpec((RB, D), lambda i: (i, 0)),
            pl.BlockSpec((D, D), lambda i: (0, 0)),
            pl.BlockSpec((1, D), lambda i: (0, 0)),
        ],
        out_specs=pl.BlockSpec((RB, D), lambda i: (i, 0)),
        out_shape=jax.ShapeDtypeStruct((N, D), jnp.float32),
    )(x, wt, b2)


def _elem_body(a_ref, b_ref, wf_ref, dg_ref, o_ref):
    dg = dg_ref[...]
    dg = jnp.where(dg == 0.0, 1.0, dg)
    t2 = (a_ref[...] + b_ref[...]) / dg
    wfv = wf_ref[...]
    o_ref[...] = wfv * (t2 - wfv)


def _elem(a, b, wf, dg2):
    return pl.pallas_call(
        _elem_body,
        grid=(N // RB,),
        in_specs=[
            pl.BlockSpec((RB, D), lambda i: (i, 0)),
            pl.BlockSpec((RB, D), lambda i: (i, 0)),
            pl.BlockSpec((RB, D), lambda i: (i, 0)),
            pl.BlockSpec((RB, 1), lambda i: (i, 0)),
        ],
        out_specs=pl.BlockSpec((RB, D), lambda i: (i, 0)),
        out_shape=jax.ShapeDtypeStruct((N, D), jnp.float32),
    )(a, b, wf, dg2)


def _add_body(a_ref, b_ref, o_ref):
    o_ref[...] = a_ref[...] + b_ref[...]


def _combine(a, b):
    return pl.pallas_call(
        _add_body,
        grid=(N // RB,),
        in_specs=[
            pl.BlockSpec((RB, D), lambda i: (i, 0)),
            pl.BlockSpec((RB, D), lambda i: (i, 0)),
        ],
        out_specs=pl.BlockSpec((RB, D), lambda i: (i, 0)),
        out_shape=jax.ShapeDtypeStruct((N, D), jnp.float32),
    )(a, b)


def kernel(node_features, edge_index, edge_values, degree, W, b):
    row = edge_index[0].astype(jnp.int32)
    col = edge_index[1].astype(jnp.int32)
    ev = edge_values.astype(jnp.float32)
    deg = degree.astype(jnp.float32)

    wf = _linear(node_features, W.T, b.reshape(1, D))
    vu, sp = _prep1(row, col, ev, deg)
    vb = _prep2(sp, deg, row, vu)

    t1p = _spmm(vu, row, col, wf)
    temp1 = _combine(t1p[0, :N], t1p[1, :N])
    t2p = _spmm(vb, row, col, temp1)
    sh = _elem(t2p[0, :N], t2p[1, :N], wf, deg.reshape(N, 1))
    op = _spmm(ev, row, col, sh)
    return _combine(op[0, :N], op[1, :N])
